# Initial kernel scaffold; baseline (speedup 1.0000x reference)
import jax
import jax.numpy as jnp
from jax.experimental import pallas as pl


def kernel(node_attr, edge_attr, edge_index, params):
    raise NotImplementedError("write your pallas kernel here")


# speedup vs baseline: 3.2215x; 3.2215x over previous
import functools

import jax
import jax.numpy as jnp
from jax import lax
from jax.experimental import pallas as pl
from jax.experimental.pallas import tpu as pltpu
from jax.experimental.pallas import tpu_sc as plsc

N = 10000
E = 320000
D = 128
OUT = 4
NB = 2000
EB = 3200

NC = 2
NS = 16
NW = NC * NS
EW = E // NW
GK = 400
SK = 200
ROWS = N // NS


def _ln(y, g, b):
    mu = jnp.mean(y, axis=-1, keepdims=True)
    var = jnp.mean((y - mu) ** 2, axis=-1, keepdims=True)
    return (y - mu) * lax.rsqrt(var + 1e-5) * g + b


def _gf_body(na, w1, b1, w2, b2, out):
    h = jnp.maximum(na[...] @ w1[...] + b1[...], 0.0)
    y = h @ w2[...] + b2[...]
    s = jnp.sum(y, axis=0, keepdims=True) * (1.0 / N)

    @pl.when(pl.program_id(0) == 0)
    def _():
        out[...] = s

    @pl.when(pl.program_id(0) != 0)
    def _():
        out[...] += s


def _tc_global_mean(na, w1, b1, w2, b2):
    w = lambda shape: pl.BlockSpec(shape, lambda i: (0, 0))
    return pl.pallas_call(
        _gf_body,
        grid=(N // NB,),
        in_specs=[pl.BlockSpec((NB, D), lambda i: (i, 0)),
                  w((D, D)), w((1, D)), w((D, D)), w((1, D))],
        out_specs=pl.BlockSpec((1, D), lambda i: (0, 0)),
        out_shape=jax.ShapeDtypeStruct((1, D), jnp.float32),
    )(na, w1, b1, w2, b2)


def _node_enc_body(na, gf, w1a, w1g, b1, w2, b2, g, bt, out):
    h = jnp.maximum(na[...] @ w1a[...] + gf[...] @ w1g[...] + b1[...], 0.0)
    y = h @ w2[...] + b2[...]
    out[...] = _ln(y, g[...], bt[...])


def _tc_node_enc(na, gf, w1a, w1g, b1, w2, b2, g, bt):
    w = lambda shape: pl.BlockSpec(shape, lambda i: (0, 0))
    return pl.pallas_call(
        _node_enc_body,
        grid=(N // NB,),
        in_specs=[pl.BlockSpec((NB, D), lambda i: (i, 0)), w((1, D)),
                  w((D, D)), w((D, D)), w((1, D)), w((D, D)), w((1, D)),
                  w((1, D)), w((1, D))],
        out_specs=pl.BlockSpec((NB, D), lambda i: (i, 0)),
        out_shape=jax.ShapeDtypeStruct((N, D), jnp.float32),
    )(na, gf, w1a, w1g, b1, w2, b2, g, bt)


def _edge_enc_body(ea, w1, b1, w2, b2, g, bt, out):
    h = jnp.maximum(ea[...] @ w1[...] + b1[...], 0.0)
    y = h @ w2[...] + b2[...]
    out[...] = _ln(y, g[...], bt[...])


def _tc_edge_enc(ea, w1, b1, w2, b2, g, bt):
    w = lambda shape: pl.BlockSpec(shape, lambda i: (0, 0))
    return pl.pallas_call(
        _edge_enc_body,
        grid=(E // EB,),
        in_specs=[pl.BlockSpec((EB, 16), lambda i: (i, 0)),
                  w((16, D)), w((1, D)), w((D, D)), w((1, D)),
                  w((1, D)), w((1, D))],
        out_specs=pl.BlockSpec((EB, D), lambda i: (i, 0)),
        out_shape=jax.ShapeDtypeStruct((E, D), jnp.float32),
    )(ea, w1, b1, w2, b2, g, bt)


def _uv_body(x, a, b, b1e, u, v):
    xv = x[...]
    u[...] = xv @ a[...] + b1e[...]
    v[...] = xv @ b[...]


def _tc_uv(x, a, b, b1e):
    w = lambda shape: pl.BlockSpec(shape, lambda i: (0, 0))
    return pl.pallas_call(
        _uv_body,
        grid=(N // NB,),
        in_specs=[pl.BlockSpec((NB, D), lambda i: (i, 0)),
                  w((D, D)), w((D, D)), w((1, D))],
        out_specs=[pl.BlockSpec((NB, D), lambda i: (i, 0)),
                   pl.BlockSpec((NB, D), lambda i: (i, 0))],
        out_shape=[jax.ShapeDtypeStruct((N, D), jnp.float32),
                   jax.ShapeDtypeStruct((N, D), jnp.float32)],
    )(x, a, b, b1e)


def _edge_body(g1, g2, e, c, w2, b2, g, bt, out):
    ev = e[...]
    h = jnp.maximum(g1[...] + g2[...] + ev @ c[...], 0.0)
    y = h @ w2[...] + b2[...]
    out[...] = _ln(y, g[...], bt[...]) + ev


def _tc_edge(g1, g2, e, c, w2, b2, g, bt):
    w = lambda shape: pl.BlockSpec(shape, lambda i: (0, 0))
    eb = pl.BlockSpec((EB, D), lambda i: (i, 0))
    return pl.pallas_call(
        _edge_body,
        grid=(E // EB,),
        in_specs=[eb, eb, eb, w((D, D)), w((D, D)), w((1, D)),
                  w((1, D)), w((1, D))],
        out_specs=eb,
        out_shape=jax.ShapeDtypeStruct((E, D), jnp.float32),
    )(g1, g2, e, c, w2, b2, g, bt)


def _node_body(x, a0, a1, wx, wa, b1, w2, b2, g, bt, out):
    xv = x[...]
    agg = a0[...] + a1[...]
    h = jnp.maximum(xv @ wx[...] + agg @ wa[...] + b1[...], 0.0)
    y = h @ w2[...] + b2[...]
    out[...] = _ln(y, g[...], bt[...]) + xv


def _tc_node(x, aggs, wx, wa, b1, w2, b2, g, bt):
    w = lambda shape: pl.BlockSpec(shape, lambda i: (0, 0))
    nb = pl.BlockSpec((NB, D), lambda i: (i, 0))
    nb1 = pl.BlockSpec((NB, D), lambda i: (N // NB + i, 0))
    return pl.pallas_call(
        _node_body,
        grid=(N // NB,),
        in_specs=[nb, nb, nb1, w((D, D)), w((D, D)), w((1, D)), w((D, D)),
                  w((1, D)), w((1, D)), w((1, D))],
        out_specs=nb,
        out_shape=jax.ShapeDtypeStruct((N, D), jnp.float32),
    )(x, aggs, aggs, wx, wa, b1, w2, b2, g, bt)


def _dec_body(x, w1, b1, w2, b2, out):
    h = jnp.maximum(x[...] @ w1[...] + b1[...], 0.0)
    out[...] = h @ w2[...] + b2[...]


def _tc_dec(x, w1, b1, w2, b2):
    w = lambda shape: pl.BlockSpec(shape, lambda i: (0, 0))
    return pl.pallas_call(
        _dec_body,
        grid=(N // NB,),
        in_specs=[pl.BlockSpec((NB, D), lambda i: (i, 0)),
                  w((D, D)), w((1, D)), w((D, OUT)), w((1, OUT))],
        out_specs=pl.BlockSpec((NB, OUT), lambda i: (i, 0)),
        out_shape=jax.ShapeDtypeStruct((N, OUT), jnp.float32),
    )(x, w1, b1, w2, b2)


@functools.cache
def _sc_gather_kernel():
    mesh = plsc.VectorSubcoreMesh(core_axis_name="c", subcore_axis_name="s")

    @functools.partial(
        pl.kernel,
        out_type=(jax.ShapeDtypeStruct((E, D), jnp.float32),
                  jax.ShapeDtypeStruct((E, D), jnp.float32)),
        mesh=mesh,
        scratch_types=[
            pltpu.VMEM((GK,), jnp.int32),
            pltpu.VMEM((GK,), jnp.int32),
            pltpu.VMEM((GK, D), jnp.float32),
            pltpu.VMEM((GK, D), jnp.float32),
            pltpu.SemaphoreType.DMA,
            pltpu.SemaphoreType.DMA,
            pltpu.SemaphoreType.DMA,
            pltpu.SemaphoreType.DMA,
        ],
    )
    def gather(u_hbm, v_hbm, src_hbm, dst_hbm, g1_hbm, g2_hbm,
               sidx, didx, bu, bv, s1, s2, s3, s4):
        wid = lax.axis_index("s") * NC + lax.axis_index("c")
        base = wid * EW

        def chunk(i, carry):
            off = base + i * GK
            pltpu.sync_copy(src_hbm.at[pl.ds(off, GK)], sidx)
            pltpu.sync_copy(dst_hbm.at[pl.ds(off, GK)], didx)
            c1 = pltpu.async_copy(u_hbm.at[sidx], bu, s1)
            c2 = pltpu.async_copy(v_hbm.at[didx], bv, s2)
            c1.wait()
            c2.wait()
            c3 = pltpu.async_copy(bu, g1_hbm.at[pl.ds(off, GK)], s3)
            c4 = pltpu.async_copy(bv, g2_hbm.at[pl.ds(off, GK)], s4)
            c3.wait()
            c4.wait()
            return carry

        lax.fori_loop(0, EW // GK, chunk, 0)

    return gather


def _sc_gather(u, v, src, dst):
    return _sc_gather_kernel()(u, v, src, dst)


@functools.cache
def _sc_scatter_kernel():
    mesh = plsc.VectorSubcoreMesh(core_axis_name="c", subcore_axis_name="s")

    @functools.partial(
        pl.kernel,
        out_type=jax.ShapeDtypeStruct((NC, NS, ROWS, D), jnp.float32),
        mesh=mesh,
        scratch_types=[
            pltpu.VMEM_SHARED((N, D), jnp.float32),
            pltpu.VMEM((SK, D), jnp.float32),
            pltpu.VMEM((SK,), jnp.int32),
        ],
    )
    def scatter(e_hbm, dst_hbm, zeros_hbm, out_hbm, shared, buf, didx):
        c = lax.axis_index("c")
        s = lax.axis_index("s")
        wid = s * NC + c
        r0 = s * ROWS
        pltpu.sync_copy(zeros_hbm, shared.at[pl.ds(r0, ROWS)])
        plsc.subcore_barrier()
        base = wid * EW

        def chunk(i, carry):
            off = base + i * SK
            pltpu.sync_copy(dst_hbm.at[pl.ds(off, SK)], didx)
            pltpu.sync_copy(e_hbm.at[pl.ds(off, SK)], buf)
            pltpu.sync_copy(buf, shared.at[didx], add=True)
            return carry

        lax.fori_loop(0, EW // SK, chunk, 0)
        plsc.subcore_barrier()
        pltpu.sync_copy(shared.at[pl.ds(r0, ROWS)], out_hbm.at[c, s])

    return scatter


def _sc_scatter(e2, dst, zeros_rows):
    out = _sc_scatter_kernel()(e2, dst, zeros_rows)
    return out.reshape(NC * N, D)


def _rowvec(b):
    return b.reshape(1, -1)


def kernel(node_attr, edge_attr, edge_index, params):
    src = edge_index[0]
    dst = edge_index[1]
    p = params
    zeros_rows = jnp.zeros((ROWS, D), jnp.float32)

    ge = p['global_enc']
    gf = _tc_global_mean(node_attr, ge['W1'], _rowvec(ge['b1']),
                         ge['W2'], _rowvec(ge['b2']))

    ne = p['node_enc']
    x = _tc_node_enc(node_attr, gf, ne['W1'][:D], ne['W1'][D:],
                     _rowvec(ne['b1']), ne['W2'], _rowvec(ne['b2']),
                     _rowvec(ne['g']), _rowvec(ne['bt']))

    ee = p['edge_enc']
    e = _tc_edge_enc(edge_attr, ee['W1'], _rowvec(ee['b1']), ee['W2'],
                     _rowvec(ee['b2']), _rowvec(ee['g']), _rowvec(ee['bt']))

    def stk(fn):
        return jnp.stack([fn(lp) for lp in p['layers']])

    ws = {
        'A': stk(lambda lp: lp['edge_mlp']['W1'][:D]),
        'B': stk(lambda lp: lp['edge_mlp']['W1'][D:2 * D]),
        'C': stk(lambda lp: lp['edge_mlp']['W1'][2 * D:]),
        'b1e': stk(lambda lp: _rowvec(lp['edge_mlp']['b1'])),
        'W2e': stk(lambda lp: lp['edge_mlp']['W2']),
        'b2e': stk(lambda lp: _rowvec(lp['edge_mlp']['b2'])),
        'ge': stk(lambda lp: _rowvec(lp['edge_mlp']['g'])),
        'be': stk(lambda lp: _rowvec(lp['edge_mlp']['bt'])),
        'Wx': stk(lambda lp: lp['node_mlp']['W1'][:D]),
        'Wa': stk(lambda lp: lp['node_mlp']['W1'][D:]),
        'b1n': stk(lambda lp: _rowvec(lp['node_mlp']['b1'])),
        'W2n': stk(lambda lp: lp['node_mlp']['W2']),
        'b2n': stk(lambda lp: _rowvec(lp['node_mlp']['b2'])),
        'gn': stk(lambda lp: _rowvec(lp['node_mlp']['g'])),
        'bn': stk(lambda lp: _rowvec(lp['node_mlp']['bt'])),
    }

    def body(carry, w):
        x, e = carry
        u, v = _tc_uv(x, w['A'], w['B'], w['b1e'])
        g1, g2 = _sc_gather(u, v, src, dst)
        e2 = _tc_edge(g1, g2, e, w['C'], w['W2e'], w['b2e'],
                      w['ge'], w['be'])
        aggs = _sc_scatter(e2, dst, zeros_rows)
        x2 = _tc_node(x, aggs, w['Wx'], w['Wa'], w['b1n'], w['W2n'],
                      w['b2n'], w['gn'], w['bn'])
        return (x2, e2), None

    (x, e), _ = lax.scan(body, (x, e), ws)

    dec = p['decoder']
    return _tc_dec(x, dec['W1'], _rowvec(dec['b1']), dec['W2'],
                   _rowvec(dec['b2']))


# speedup vs baseline: 3.3566x; 1.0419x over previous
import functools

import jax
import jax.numpy as jnp
from jax import lax
from jax.experimental import pallas as pl
from jax.experimental.pallas import tpu as pltpu
from jax.experimental.pallas import tpu_sc as plsc

N = 10000
E = 320000
D = 128
OUT = 4
NB = 2000
EB = 3200

NC = 2
NS = 16
NW = NC * NS
EW = E // NW
GK = 200
SK = 40
ROWS = N // NS


def _ln(y, g, b):
    mu = jnp.mean(y, axis=-1, keepdims=True)
    var = jnp.mean((y - mu) ** 2, axis=-1, keepdims=True)
    return (y - mu) * lax.rsqrt(var + 1e-5) * g + b


def _gf_body(na, w1, b1, w2, b2, out):
    h = jnp.maximum(na[...] @ w1[...] + b1[...], 0.0)
    y = h @ w2[...] + b2[...]
    s = jnp.sum(y, axis=0, keepdims=True) * (1.0 / N)

    @pl.when(pl.program_id(0) == 0)
    def _():
        out[...] = s

    @pl.when(pl.program_id(0) != 0)
    def _():
        out[...] += s


def _tc_global_mean(na, w1, b1, w2, b2):
    w = lambda shape: pl.BlockSpec(shape, lambda i: (0, 0))
    return pl.pallas_call(
        _gf_body,
        grid=(N // NB,),
        in_specs=[pl.BlockSpec((NB, D), lambda i: (i, 0)),
                  w((D, D)), w((1, D)), w((D, D)), w((1, D))],
        out_specs=pl.BlockSpec((1, D), lambda i: (0, 0)),
        out_shape=jax.ShapeDtypeStruct((1, D), jnp.float32),
    )(na, w1, b1, w2, b2)


def _node_enc_body(na, gf, w1a, w1g, b1, w2, b2, g, bt, out):
    h = jnp.maximum(na[...] @ w1a[...] + gf[...] @ w1g[...] + b1[...], 0.0)
    y = h @ w2[...] + b2[...]
    out[...] = _ln(y, g[...], bt[...])


def _tc_node_enc(na, gf, w1a, w1g, b1, w2, b2, g, bt):
    w = lambda shape: pl.BlockSpec(shape, lambda i: (0, 0))
    return pl.pallas_call(
        _node_enc_body,
        grid=(N // NB,),
        in_specs=[pl.BlockSpec((NB, D), lambda i: (i, 0)), w((1, D)),
                  w((D, D)), w((D, D)), w((1, D)), w((D, D)), w((1, D)),
                  w((1, D)), w((1, D))],
        out_specs=pl.BlockSpec((NB, D), lambda i: (i, 0)),
        out_shape=jax.ShapeDtypeStruct((N, D), jnp.float32),
    )(na, gf, w1a, w1g, b1, w2, b2, g, bt)


def _edge_enc_body(ea, w1, b1, w2, b2, g, bt, out):
    h = jnp.maximum(ea[...] @ w1[...] + b1[...], 0.0)
    y = h @ w2[...] + b2[...]
    out[...] = _ln(y, g[...], bt[...])


def _tc_edge_enc(ea, w1, b1, w2, b2, g, bt):
    w = lambda shape: pl.BlockSpec(shape, lambda i: (0, 0))
    return pl.pallas_call(
        _edge_enc_body,
        grid=(E // EB,),
        in_specs=[pl.BlockSpec((EB, 16), lambda i: (i, 0)),
                  w((16, D)), w((1, D)), w((D, D)), w((1, D)),
                  w((1, D)), w((1, D))],
        out_specs=pl.BlockSpec((EB, D), lambda i: (i, 0)),
        out_shape=jax.ShapeDtypeStruct((E, D), jnp.float32),
    )(ea, w1, b1, w2, b2, g, bt)


def _uv_body(x, a, b, b1e, u, v):
    xv = x[...]
    u[...] = xv @ a[...] + b1e[...]
    v[...] = xv @ b[...]


def _tc_uv(x, a, b, b1e):
    w = lambda shape: pl.BlockSpec(shape, lambda i: (0, 0))
    return pl.pallas_call(
        _uv_body,
        grid=(N // NB,),
        in_specs=[pl.BlockSpec((NB, D), lambda i: (i, 0)),
                  w((D, D)), w((D, D)), w((1, D))],
        out_specs=[pl.BlockSpec((NB, D), lambda i: (i, 0)),
                   pl.BlockSpec((NB, D), lambda i: (i, 0))],
        out_shape=[jax.ShapeDtypeStruct((N, D), jnp.float32),
                   jax.ShapeDtypeStruct((N, D), jnp.float32)],
    )(x, a, b, b1e)


def _edge_body(g1, g2, e, c, w2, b2, g, bt, out):
    ev = e[...]
    h = jnp.maximum(g1[...] + g2[...] + ev @ c[...], 0.0)
    y = h @ w2[...] + b2[...]
    out[...] = _ln(y, g[...], bt[...]) + ev


def _tc_edge(g1, g2, e, c, w2, b2, g, bt):
    w = lambda shape: pl.BlockSpec(shape, lambda i: (0, 0))
    eb = pl.BlockSpec((EB, D), lambda i: (i, 0))
    return pl.pallas_call(
        _edge_body,
        grid=(E // EB,),
        in_specs=[eb, eb, eb, w((D, D)), w((D, D)), w((1, D)),
                  w((1, D)), w((1, D))],
        out_specs=eb,
        out_shape=jax.ShapeDtypeStruct((E, D), jnp.float32),
    )(g1, g2, e, c, w2, b2, g, bt)


def _node_body(x, a0, a1, wx, wa, b1, w2, b2, g, bt, out):
    xv = x[...]
    agg = a0[...] + a1[...]
    h = jnp.maximum(xv @ wx[...] + agg @ wa[...] + b1[...], 0.0)
    y = h @ w2[...] + b2[...]
    out[...] = _ln(y, g[...], bt[...]) + xv


def _tc_node(x, aggs, wx, wa, b1, w2, b2, g, bt):
    w = lambda shape: pl.BlockSpec(shape, lambda i: (0, 0))
    nb = pl.BlockSpec((NB, D), lambda i: (i, 0))
    nb1 = pl.BlockSpec((NB, D), lambda i: (N // NB + i, 0))
    return pl.pallas_call(
        _node_body,
        grid=(N // NB,),
        in_specs=[nb, nb, nb1, w((D, D)), w((D, D)), w((1, D)), w((D, D)),
                  w((1, D)), w((1, D)), w((1, D))],
        out_specs=nb,
        out_shape=jax.ShapeDtypeStruct((N, D), jnp.float32),
    )(x, aggs, aggs, wx, wa, b1, w2, b2, g, bt)


def _dec_body(x, w1, b1, w2, b2, out):
    h = jnp.maximum(x[...] @ w1[...] + b1[...], 0.0)
    out[...] = h @ w2[...] + b2[...]


def _tc_dec(x, w1, b1, w2, b2):
    w = lambda shape: pl.BlockSpec(shape, lambda i: (0, 0))
    return pl.pallas_call(
        _dec_body,
        grid=(N // NB,),
        in_specs=[pl.BlockSpec((NB, D), lambda i: (i, 0)),
                  w((D, D)), w((1, D)), w((D, OUT)), w((1, OUT))],
        out_specs=pl.BlockSpec((NB, OUT), lambda i: (i, 0)),
        out_shape=jax.ShapeDtypeStruct((N, OUT), jnp.float32),
    )(x, w1, b1, w2, b2)


@functools.cache
def _sc_gather_kernel():
    mesh = plsc.VectorSubcoreMesh(core_axis_name="c", subcore_axis_name="s")

    @functools.partial(
        pl.kernel,
        out_type=(jax.ShapeDtypeStruct((E, D), jnp.float32),
                  jax.ShapeDtypeStruct((E, D), jnp.float32)),
        mesh=mesh,
        scratch_types=[
            pltpu.VMEM((GK,), jnp.int32),
            pltpu.VMEM((GK,), jnp.int32),
            pltpu.VMEM((GK,), jnp.int32),
            pltpu.VMEM((GK,), jnp.int32),
            pltpu.VMEM((GK, D), jnp.float32),
            pltpu.VMEM((GK, D), jnp.float32),
            pltpu.VMEM((GK, D), jnp.float32),
            pltpu.VMEM((GK, D), jnp.float32),
            pltpu.SemaphoreType.DMA,
            pltpu.SemaphoreType.DMA,
            pltpu.SemaphoreType.DMA,
            pltpu.SemaphoreType.DMA,
        ],
    )
    def gather(u_hbm, v_hbm, src_hbm, dst_hbm, g1_hbm, g2_hbm,
               sidx0, sidx1, didx0, didx1, bu0, bu1, bv0, bv1,
               sg0, sg1, sw0, sw1):
        wid = lax.axis_index("s") * NC + lax.axis_index("c")
        base = wid * EW
        CH = EW // GK
        bufs = {0: (sidx0, didx0, bu0, bv0, sg0, sw0),
                1: (sidx1, didx1, bu1, bv1, sg1, sw1)}

        def start_chunk(i, b):
            sidx, didx, bu, bv, sg, _ = bufs[b]
            off = base + i * GK
            pltpu.sync_copy(src_hbm.at[pl.ds(off, GK)], sidx)
            pltpu.sync_copy(dst_hbm.at[pl.ds(off, GK)], didx)
            pltpu.async_copy(u_hbm.at[sidx], bu, sg)
            pltpu.async_copy(v_hbm.at[didx], bv, sg)

        def wait_gather(b):
            _, _, bu, bv, sg, _ = bufs[b]
            pltpu.make_async_copy(g1_hbm.at[pl.ds(0, GK)], bu, sg).wait()
            pltpu.make_async_copy(g1_hbm.at[pl.ds(0, GK)], bv, sg).wait()

        def start_write(i, b):
            _, _, bu, bv, _, sw = bufs[b]
            off = base + i * GK
            pltpu.async_copy(bu, g1_hbm.at[pl.ds(off, GK)], sw)
            pltpu.async_copy(bv, g2_hbm.at[pl.ds(off, GK)], sw)

        def wait_write(b):
            _, _, bu, bv, _, sw = bufs[b]
            pltpu.make_async_copy(bu, g1_hbm.at[pl.ds(0, GK)], sw).wait()
            pltpu.make_async_copy(bv, g2_hbm.at[pl.ds(0, GK)], sw).wait()

        start_chunk(0, 0)
        wait_gather(0)
        start_write(0, 0)
        start_chunk(1, 1)

        def pair(j, carry):
            i1 = 1 + 2 * j
            wait_gather(1)
            start_write(i1, 1)
            wait_write(0)
            start_chunk(i1 + 1, 0)
            i2 = i1 + 1
            wait_gather(0)
            start_write(i2, 0)
            wait_write(1)
            start_chunk(i2 + 1, 1)
            return carry

        lax.fori_loop(0, (CH - 2) // 2, pair, 0)
        wait_gather(1)
        start_write(CH - 1, 1)
        wait_write(0)
        wait_write(1)

    return gather


def _sc_gather(u, v, src, dst):
    return _sc_gather_kernel()(u, v, src, dst)


@functools.cache
def _sc_scatter_kernel():
    mesh = plsc.VectorSubcoreMesh(core_axis_name="c", subcore_axis_name="s")

    @functools.partial(
        pl.kernel,
        out_type=jax.ShapeDtypeStruct((NC, NS, ROWS, D), jnp.float32),
        mesh=mesh,
        scratch_types=[
            pltpu.VMEM_SHARED((N, D), jnp.float32),
            pltpu.VMEM((SK, D), jnp.float32),
            pltpu.VMEM((SK, D), jnp.float32),
            pltpu.VMEM((SK,), jnp.int32),
            pltpu.VMEM((SK,), jnp.int32),
            pltpu.SemaphoreType.DMA,
            pltpu.SemaphoreType.DMA,
        ],
    )
    def scatter(e_hbm, dst_hbm, zeros_hbm, out_hbm, shared,
                buf0, buf1, didx0, didx1, sl0, sl1):
        c = lax.axis_index("c")
        s = lax.axis_index("s")
        wid = s * NC + c
        r0 = s * ROWS
        pltpu.sync_copy(zeros_hbm, shared.at[pl.ds(r0, ROWS)])
        plsc.subcore_barrier()
        base = wid * EW
        CH = EW // SK
        bufs = {0: (buf0, didx0, sl0), 1: (buf1, didx1, sl1)}

        def start_load(i, b):
            buf, didx, sl = bufs[b]
            off = base + i * SK
            pltpu.async_copy(dst_hbm.at[pl.ds(off, SK)], didx, sl)
            pltpu.async_copy(e_hbm.at[pl.ds(off, SK)], buf, sl)

        def scatter_chunk(b):
            buf, didx, sl = bufs[b]
            pltpu.make_async_copy(dst_hbm.at[pl.ds(0, SK)], didx, sl).wait()
            pltpu.make_async_copy(e_hbm.at[pl.ds(0, SK)], buf, sl).wait()
            pltpu.sync_copy(buf, shared.at[didx], add=True)

        start_load(0, 0)

        def pair(j, carry):
            start_load(2 * j + 1, 1)
            scatter_chunk(0)
            start_load(2 * j + 2, 0)
            scatter_chunk(1)
            return carry

        lax.fori_loop(0, (CH - 2) // 2, pair, 0)
        start_load(CH - 1, 1)
        scatter_chunk(0)
        scatter_chunk(1)
        plsc.subcore_barrier()
        pltpu.sync_copy(shared.at[pl.ds(r0, ROWS)], out_hbm.at[c, s])

    return scatter


def _sc_scatter(e2, dst, zeros_rows):
    out = _sc_scatter_kernel()(e2, dst, zeros_rows)
    return out.reshape(NC * N, D)


def _rowvec(b):
    return b.reshape(1, -1)


def kernel(node_attr, edge_attr, edge_index, params):
    src = edge_index[0]
    dst = edge_index[1]
    p = params
    zeros_rows = jnp.zeros((ROWS, D), jnp.float32)

    ge = p['global_enc']
    gf = _tc_global_mean(node_attr, ge['W1'], _rowvec(ge['b1']),
                         ge['W2'], _rowvec(ge['b2']))

    ne = p['node_enc']
    x = _tc_node_enc(node_attr, gf, ne['W1'][:D], ne['W1'][D:],
                     _rowvec(ne['b1']), ne['W2'], _rowvec(ne['b2']),
                     _rowvec(ne['g']), _rowvec(ne['bt']))

    ee = p['edge_enc']
    e = _tc_edge_enc(edge_attr, ee['W1'], _rowvec(ee['b1']), ee['W2'],
                     _rowvec(ee['b2']), _rowvec(ee['g']), _rowvec(ee['bt']))

    def stk(fn):
        return jnp.stack([fn(lp) for lp in p['layers']])

    ws = {
        'A': stk(lambda lp: lp['edge_mlp']['W1'][:D]),
        'B': stk(lambda lp: lp['edge_mlp']['W1'][D:2 * D]),
        'C': stk(lambda lp: lp['edge_mlp']['W1'][2 * D:]),
        'b1e': stk(lambda lp: _rowvec(lp['edge_mlp']['b1'])),
        'W2e': stk(lambda lp: lp['edge_mlp']['W2']),
        'b2e': stk(lambda lp: _rowvec(lp['edge_mlp']['b2'])),
        'ge': stk(lambda lp: _rowvec(lp['edge_mlp']['g'])),
        'be': stk(lambda lp: _rowvec(lp['edge_mlp']['bt'])),
        'Wx': stk(lambda lp: lp['node_mlp']['W1'][:D]),
        'Wa': stk(lambda lp: lp['node_mlp']['W1'][D:]),
        'b1n': stk(lambda lp: _rowvec(lp['node_mlp']['b1'])),
        'W2n': stk(lambda lp: lp['node_mlp']['W2']),
        'b2n': stk(lambda lp: _rowvec(lp['node_mlp']['b2'])),
        'gn': stk(lambda lp: _rowvec(lp['node_mlp']['g'])),
        'bn': stk(lambda lp: _rowvec(lp['node_mlp']['bt'])),
    }

    def body(carry, w):
        x, e = carry
        u, v = _tc_uv(x, w['A'], w['B'], w['b1e'])
        g1, g2 = _sc_gather(u, v, src, dst)
        e2 = _tc_edge(g1, g2, e, w['C'], w['W2e'], w['b2e'],
                      w['ge'], w['be'])
        aggs = _sc_scatter(e2, dst, zeros_rows)
        x2 = _tc_node(x, aggs, w['Wx'], w['Wa'], w['b1n'], w['W2n'],
                      w['b2n'], w['gn'], w['bn'])
        return (x2, e2), None

    (x, e), _ = lax.scan(body, (x, e), ws)

    dec = p['decoder']
    return _tc_dec(x, dec['W1'], _rowvec(dec['b1']), dec['W2'],
                   _rowvec(dec['b2']))


# speedup vs baseline: 3.3978x; 1.0123x over previous
import functools

import jax
import jax.numpy as jnp
from jax import lax
from jax.experimental import pallas as pl
from jax.experimental.pallas import tpu as pltpu
from jax.experimental.pallas import tpu_sc as plsc

N = 10000
E = 320000
D = 128
OUT = 4
NB = 2000
EB = 3200

NC = 2
NS = 16
NW = NC * NS
EW = E // NW
GK = 200
SK = 40
ROWS = N // NS


def _ln(y, g, b):
    mu = jnp.mean(y, axis=-1, keepdims=True)
    var = jnp.mean((y - mu) ** 2, axis=-1, keepdims=True)
    return (y - mu) * lax.rsqrt(var + 1e-5) * g + b


def _gf_body(na, w1, b1, w2, b2, out):
    h = jnp.maximum(na[...] @ w1[...] + b1[...], 0.0)
    y = h @ w2[...] + b2[...]
    s = jnp.sum(y, axis=0, keepdims=True) * (1.0 / N)

    @pl.when(pl.program_id(0) == 0)
    def _():
        out[...] = s

    @pl.when(pl.program_id(0) != 0)
    def _():
        out[...] += s


def _tc_global_mean(na, w1, b1, w2, b2):
    w = lambda shape: pl.BlockSpec(shape, lambda i: (0, 0))
    return pl.pallas_call(
        _gf_body,
        grid=(N // NB,),
        in_specs=[pl.BlockSpec((NB, D), lambda i: (i, 0)),
                  w((D, D)), w((1, D)), w((D, D)), w((1, D))],
        out_specs=pl.BlockSpec((1, D), lambda i: (0, 0)),
        out_shape=jax.ShapeDtypeStruct((1, D), jnp.float32),
    )(na, w1, b1, w2, b2)


def _node_enc_body(na, gf, w1a, w1g, b1, w2, b2, g, bt, out):
    h = jnp.maximum(na[...] @ w1a[...] + gf[...] @ w1g[...] + b1[...], 0.0)
    y = h @ w2[...] + b2[...]
    out[...] = _ln(y, g[...], bt[...])


def _tc_node_enc(na, gf, w1a, w1g, b1, w2, b2, g, bt):
    w = lambda shape: pl.BlockSpec(shape, lambda i: (0, 0))
    return pl.pallas_call(
        _node_enc_body,
        grid=(N // NB,),
        in_specs=[pl.BlockSpec((NB, D), lambda i: (i, 0)), w((1, D)),
                  w((D, D)), w((D, D)), w((1, D)), w((D, D)), w((1, D)),
                  w((1, D)), w((1, D))],
        out_specs=pl.BlockSpec((NB, D), lambda i: (i, 0)),
        out_shape=jax.ShapeDtypeStruct((N, D), jnp.float32),
    )(na, gf, w1a, w1g, b1, w2, b2, g, bt)


def _edge_enc_body(ea, w1, b1, w2, b2, g, bt, out):
    h = jnp.maximum(ea[...] @ w1[...] + b1[...], 0.0)
    y = h @ w2[...] + b2[...]
    out[...] = _ln(y, g[...], bt[...])


def _tc_edge_enc(ea, w1, b1, w2, b2, g, bt):
    w = lambda shape: pl.BlockSpec(shape, lambda i: (0, 0))
    return pl.pallas_call(
        _edge_enc_body,
        grid=(E // EB,),
        in_specs=[pl.BlockSpec((EB, 16), lambda i: (i, 0)),
                  w((16, D)), w((1, D)), w((D, D)), w((1, D)),
                  w((1, D)), w((1, D))],
        out_specs=pl.BlockSpec((EB, D), lambda i: (i, 0)),
        out_shape=jax.ShapeDtypeStruct((E, D), jnp.float32),
    )(ea, w1, b1, w2, b2, g, bt)


def _uv_body(x, a, b, b1e, u, v):
    xv = x[...]
    u[...] = xv @ a[...] + b1e[...]
    v[...] = xv @ b[...]


def _tc_uv(x, a, b, b1e):
    w = lambda shape: pl.BlockSpec(shape, lambda i: (0, 0))
    return pl.pallas_call(
        _uv_body,
        grid=(N // NB,),
        in_specs=[pl.BlockSpec((NB, D), lambda i: (i, 0)),
                  w((D, D)), w((D, D)), w((1, D))],
        out_specs=[pl.BlockSpec((NB, D), lambda i: (i, 0)),
                   pl.BlockSpec((NB, D), lambda i: (i, 0))],
        out_shape=[jax.ShapeDtypeStruct((N, D), jnp.float32),
                   jax.ShapeDtypeStruct((N, D), jnp.float32)],
    )(x, a, b, b1e)


def _edge_body(gs, e, c, w2, b2, g, bt, out):
    ev = e[...]
    h = jnp.maximum(gs[...] + ev @ c[...], 0.0)
    y = h @ w2[...] + b2[...]
    out[...] = _ln(y, g[...], bt[...]) + ev


def _tc_edge(gs, e, c, w2, b2, g, bt):
    w = lambda shape: pl.BlockSpec(shape, lambda i: (0, 0))
    eb = pl.BlockSpec((EB, D), lambda i: (i, 0))
    return pl.pallas_call(
        _edge_body,
        grid=(E // EB,),
        in_specs=[eb, eb, w((D, D)), w((D, D)), w((1, D)),
                  w((1, D)), w((1, D))],
        out_specs=eb,
        out_shape=jax.ShapeDtypeStruct((E, D), jnp.float32),
    )(gs, e, c, w2, b2, g, bt)


def _node_body(x, a0, a1, wx, wa, b1, w2, b2, g, bt, out):
    xv = x[...]
    agg = a0[...] + a1[...]
    h = jnp.maximum(xv @ wx[...] + agg @ wa[...] + b1[...], 0.0)
    y = h @ w2[...] + b2[...]
    out[...] = _ln(y, g[...], bt[...]) + xv


def _tc_node(x, aggs, wx, wa, b1, w2, b2, g, bt):
    w = lambda shape: pl.BlockSpec(shape, lambda i: (0, 0))
    nb = pl.BlockSpec((NB, D), lambda i: (i, 0))
    nb1 = pl.BlockSpec((NB, D), lambda i: (N // NB + i, 0))
    return pl.pallas_call(
        _node_body,
        grid=(N // NB,),
        in_specs=[nb, nb, nb1, w((D, D)), w((D, D)), w((1, D)), w((D, D)),
                  w((1, D)), w((1, D)), w((1, D))],
        out_specs=nb,
        out_shape=jax.ShapeDtypeStruct((N, D), jnp.float32),
    )(x, aggs, aggs, wx, wa, b1, w2, b2, g, bt)


def _dec_body(x, w1, b1, w2, b2, out):
    h = jnp.maximum(x[...] @ w1[...] + b1[...], 0.0)
    out[...] = h @ w2[...] + b2[...]


def _tc_dec(x, w1, b1, w2, b2):
    w = lambda shape: pl.BlockSpec(shape, lambda i: (0, 0))
    return pl.pallas_call(
        _dec_body,
        grid=(N // NB,),
        in_specs=[pl.BlockSpec((NB, D), lambda i: (i, 0)),
                  w((D, D)), w((1, D)), w((D, OUT)), w((1, OUT))],
        out_specs=pl.BlockSpec((NB, OUT), lambda i: (i, 0)),
        out_shape=jax.ShapeDtypeStruct((N, OUT), jnp.float32),
    )(x, w1, b1, w2, b2)


@functools.cache
def _sc_gather_kernel():
    mesh = plsc.VectorSubcoreMesh(core_axis_name="c", subcore_axis_name="s")

    @functools.partial(
        pl.kernel,
        out_type=jax.ShapeDtypeStruct((E, D), jnp.float32),
        mesh=mesh,
        scratch_types=[
            pltpu.VMEM((GK,), jnp.int32),
            pltpu.VMEM((GK,), jnp.int32),
            pltpu.VMEM((GK,), jnp.int32),
            pltpu.VMEM((GK,), jnp.int32),
            pltpu.VMEM((GK, D), jnp.float32),
            pltpu.VMEM((GK, D), jnp.float32),
            pltpu.VMEM((GK, D), jnp.float32),
            pltpu.VMEM((GK, D), jnp.float32),
            pltpu.SemaphoreType.DMA,
            pltpu.SemaphoreType.DMA,
            pltpu.SemaphoreType.DMA,
            pltpu.SemaphoreType.DMA,
        ],
    )
    def gather(u_hbm, v_hbm, src_hbm, dst_hbm, g_hbm,
               sidx0, sidx1, didx0, didx1, bu0, bu1, bv0, bv1,
               sg0, sg1, sw0, sw1):
        wid = lax.axis_index("s") * NC + lax.axis_index("c")
        base = wid * EW
        CH = EW // GK
        bufs = {0: (sidx0, didx0, bu0, bv0, sg0, sw0),
                1: (sidx1, didx1, bu1, bv1, sg1, sw1)}

        def start_chunk(i, b):
            sidx, didx, bu, bv, sg, _ = bufs[b]
            off = base + i * GK
            pltpu.sync_copy(src_hbm.at[pl.ds(off, GK)], sidx)
            pltpu.sync_copy(dst_hbm.at[pl.ds(off, GK)], didx)
            pltpu.async_copy(u_hbm.at[sidx], bu, sg)
            pltpu.async_copy(v_hbm.at[didx], bv, sg)

        def add_rows(b):
            _, _, bu, bv, sg, _ = bufs[b]
            pltpu.make_async_copy(g_hbm.at[pl.ds(0, GK)], bu, sg).wait()
            pltpu.make_async_copy(g_hbm.at[pl.ds(0, GK)], bv, sg).wait()

            def row(r, carry):
                for t in range(D // 16):
                    sl = pl.ds(t * 16, 16)
                    bu[r, sl] += bv[r, sl]
                return carry

            lax.fori_loop(0, GK, row, 0)

        def start_write(i, b):
            _, _, bu, _, _, sw = bufs[b]
            off = base + i * GK
            pltpu.async_copy(bu, g_hbm.at[pl.ds(off, GK)], sw)

        def wait_write(b):
            _, _, bu, _, _, sw = bufs[b]
            pltpu.make_async_copy(bu, g_hbm.at[pl.ds(0, GK)], sw).wait()

        start_chunk(0, 0)
        add_rows(0)
        start_write(0, 0)
        start_chunk(1, 1)

        def pair(j, carry):
            i1 = 1 + 2 * j
            add_rows(1)
            start_write(i1, 1)
            wait_write(0)
            start_chunk(i1 + 1, 0)
            i2 = i1 + 1
            add_rows(0)
            start_write(i2, 0)
            wait_write(1)
            start_chunk(i2 + 1, 1)
            return carry

        lax.fori_loop(0, (CH - 2) // 2, pair, 0)
        add_rows(1)
        start_write(CH - 1, 1)
        wait_write(0)
        wait_write(1)

    return gather


def _sc_gather(u, v, src, dst):
    return _sc_gather_kernel()(u, v, src, dst)


@functools.cache
def _sc_scatter_kernel():
    mesh = plsc.VectorSubcoreMesh(core_axis_name="c", subcore_axis_name="s")

    @functools.partial(
        pl.kernel,
        out_type=jax.ShapeDtypeStruct((NC, NS, ROWS, D), jnp.float32),
        mesh=mesh,
        scratch_types=[
            pltpu.VMEM_SHARED((N, D), jnp.float32),
            pltpu.VMEM((SK, D), jnp.float32),
            pltpu.VMEM((SK, D), jnp.float32),
            pltpu.VMEM((SK,), jnp.int32),
            pltpu.VMEM((SK,), jnp.int32),
            pltpu.SemaphoreType.DMA,
            pltpu.SemaphoreType.DMA,
        ],
    )
    def scatter(e_hbm, dst_hbm, zeros_hbm, out_hbm, shared,
                buf0, buf1, didx0, didx1, sl0, sl1):
        c = lax.axis_index("c")
        s = lax.axis_index("s")
        wid = s * NC + c
        r0 = s * ROWS
        pltpu.sync_copy(zeros_hbm, shared.at[pl.ds(r0, ROWS)])
        plsc.subcore_barrier()
        base = wid * EW
        CH = EW // SK
        bufs = {0: (buf0, didx0, sl0), 1: (buf1, didx1, sl1)}

        def start_load(i, b):
            buf, didx, sl = bufs[b]
            off = base + i * SK
            pltpu.async_copy(dst_hbm.at[pl.ds(off, SK)], didx, sl)
            pltpu.async_copy(e_hbm.at[pl.ds(off, SK)], buf, sl)

        def scatter_chunk(b):
            buf, didx, sl = bufs[b]
            pltpu.make_async_copy(dst_hbm.at[pl.ds(0, SK)], didx, sl).wait()
            pltpu.make_async_copy(e_hbm.at[pl.ds(0, SK)], buf, sl).wait()
            pltpu.sync_copy(buf, shared.at[didx], add=True)

        start_load(0, 0)

        def pair(j, carry):
            start_load(2 * j + 1, 1)
            scatter_chunk(0)
            start_load(2 * j + 2, 0)
            scatter_chunk(1)
            return carry

        lax.fori_loop(0, (CH - 2) // 2, pair, 0)
        start_load(CH - 1, 1)
        scatter_chunk(0)
        scatter_chunk(1)
        plsc.subcore_barrier()
        pltpu.sync_copy(shared.at[pl.ds(r0, ROWS)], out_hbm.at[c, s])

    return scatter


def _sc_scatter(e2, dst, zeros_rows):
    out = _sc_scatter_kernel()(e2, dst, zeros_rows)
    return out.reshape(NC * N, D)


def _rowvec(b):
    return b.reshape(1, -1)


def kernel(node_attr, edge_attr, edge_index, params):
    src = edge_index[0]
    dst = edge_index[1]
    p = params
    zeros_rows = jnp.zeros((ROWS, D), jnp.float32)

    ge = p['global_enc']
    gf = _tc_global_mean(node_attr, ge['W1'], _rowvec(ge['b1']),
                         ge['W2'], _rowvec(ge['b2']))

    ne = p['node_enc']
    x = _tc_node_enc(node_attr, gf, ne['W1'][:D], ne['W1'][D:],
                     _rowvec(ne['b1']), ne['W2'], _rowvec(ne['b2']),
                     _rowvec(ne['g']), _rowvec(ne['bt']))

    ee = p['edge_enc']
    e = _tc_edge_enc(edge_attr, ee['W1'], _rowvec(ee['b1']), ee['W2'],
                     _rowvec(ee['b2']), _rowvec(ee['g']), _rowvec(ee['bt']))

    def stk(fn):
        return jnp.stack([fn(lp) for lp in p['layers']])

    ws = {
        'A': stk(lambda lp: lp['edge_mlp']['W1'][:D]),
        'B': stk(lambda lp: lp['edge_mlp']['W1'][D:2 * D]),
        'C': stk(lambda lp: lp['edge_mlp']['W1'][2 * D:]),
        'b1e': stk(lambda lp: _rowvec(lp['edge_mlp']['b1'])),
        'W2e': stk(lambda lp: lp['edge_mlp']['W2']),
        'b2e': stk(lambda lp: _rowvec(lp['edge_mlp']['b2'])),
        'ge': stk(lambda lp: _rowvec(lp['edge_mlp']['g'])),
        'be': stk(lambda lp: _rowvec(lp['edge_mlp']['bt'])),
        'Wx': stk(lambda lp: lp['node_mlp']['W1'][:D]),
        'Wa': stk(lambda lp: lp['node_mlp']['W1'][D:]),
        'b1n': stk(lambda lp: _rowvec(lp['node_mlp']['b1'])),
        'W2n': stk(lambda lp: lp['node_mlp']['W2']),
        'b2n': stk(lambda lp: _rowvec(lp['node_mlp']['b2'])),
        'gn': stk(lambda lp: _rowvec(lp['node_mlp']['g'])),
        'bn': stk(lambda lp: _rowvec(lp['node_mlp']['bt'])),
    }

    def body(carry, w):
        x, e = carry
        u, v = _tc_uv(x, w['A'], w['B'], w['b1e'])
        gs = _sc_gather(u, v, src, dst)
        e2 = _tc_edge(gs, e, w['C'], w['W2e'], w['b2e'],
                      w['ge'], w['be'])
        aggs = _sc_scatter(e2, dst, zeros_rows)
        x2 = _tc_node(x, aggs, w['Wx'], w['Wa'], w['b1n'], w['W2n'],
                      w['b2n'], w['gn'], w['bn'])
        return (x2, e2), None

    (x, e), _ = lax.scan(body, (x, e), ws)

    dec = p['decoder']
    return _tc_dec(x, dec['W1'], _rowvec(dec['b1']), dec['W2'],
                   _rowvec(dec['b2']))


# speedup vs baseline: 3.8174x; 1.1235x over previous
import functools

import jax
import jax.numpy as jnp
from jax import lax
from jax.experimental import pallas as pl
from jax.experimental.pallas import tpu as pltpu
from jax.experimental.pallas import tpu_sc as plsc

N = 10000
E = 320000
D = 128
OUT = 4
NB = 2000
EB = 3200

NC = 2
NS = 16
NW = NC * NS
EHALF = (153600, 166400)
ELO = (0, 153600)
GK = 200
SK = 40
ROWS = N // NS


def _ln(y, g, b):
    mu = jnp.mean(y, axis=-1, keepdims=True)
    var = jnp.mean((y - mu) ** 2, axis=-1, keepdims=True)
    return (y - mu) * lax.rsqrt(var + 1e-5) * g + b


def _gf_body(na, w1, b1, w2, b2, out):
    h = jnp.maximum(na[...] @ w1[...] + b1[...], 0.0)
    y = h @ w2[...] + b2[...]
    s = jnp.sum(y, axis=0, keepdims=True) * (1.0 / N)

    @pl.when(pl.program_id(0) == 0)
    def _():
        out[...] = s

    @pl.when(pl.program_id(0) != 0)
    def _():
        out[...] += s


def _tc_global_mean(na, w1, b1, w2, b2):
    w = lambda shape: pl.BlockSpec(shape, lambda i: (0, 0))
    return pl.pallas_call(
        _gf_body,
        grid=(N // NB,),
        in_specs=[pl.BlockSpec((NB, D), lambda i: (i, 0)),
                  w((D, D)), w((1, D)), w((D, D)), w((1, D))],
        out_specs=pl.BlockSpec((1, D), lambda i: (0, 0)),
        out_shape=jax.ShapeDtypeStruct((1, D), jnp.float32),
    )(na, w1, b1, w2, b2)


def _node_enc_body(na, gf, w1a, w1g, b1, w2, b2, g, bt, out):
    h = jnp.maximum(na[...] @ w1a[...] + gf[...] @ w1g[...] + b1[...], 0.0)
    y = h @ w2[...] + b2[...]
    out[...] = _ln(y, g[...], bt[...])


def _tc_node_enc(na, gf, w1a, w1g, b1, w2, b2, g, bt):
    w = lambda shape: pl.BlockSpec(shape, lambda i: (0, 0))
    return pl.pallas_call(
        _node_enc_body,
        grid=(N // NB,),
        in_specs=[pl.BlockSpec((NB, D), lambda i: (i, 0)), w((1, D)),
                  w((D, D)), w((D, D)), w((1, D)), w((D, D)), w((1, D)),
                  w((1, D)), w((1, D))],
        out_specs=pl.BlockSpec((NB, D), lambda i: (i, 0)),
        out_shape=jax.ShapeDtypeStruct((N, D), jnp.float32),
    )(na, gf, w1a, w1g, b1, w2, b2, g, bt)


def _edge_enc_body(ea, w1, b1, w2, b2, g, bt, out):
    h = jnp.maximum(ea[...] @ w1[...] + b1[...], 0.0)
    y = h @ w2[...] + b2[...]
    out[...] = _ln(y, g[...], bt[...])


def _tc_edge_enc(ea, w1, b1, w2, b2, g, bt, h):
    w = lambda shape: pl.BlockSpec(shape, lambda i: (0, 0))
    offb = ELO[h] // EB
    return pl.pallas_call(
        _edge_enc_body,
        grid=(EHALF[h] // EB,),
        in_specs=[pl.BlockSpec((EB, 16), lambda i: (i + offb, 0)),
                  w((16, D)), w((1, D)), w((D, D)), w((1, D)),
                  w((1, D)), w((1, D))],
        out_specs=pl.BlockSpec((EB, D), lambda i: (i, 0)),
        out_shape=jax.ShapeDtypeStruct((EHALF[h], D), jnp.float32),
    )(ea, w1, b1, w2, b2, g, bt)


def _uv_body(x, a, b, b1e, u, v):
    xv = x[...]
    u[...] = xv @ a[...] + b1e[...]
    v[...] = xv @ b[...]


def _tc_uv(x, a, b, b1e):
    w = lambda shape: pl.BlockSpec(shape, lambda i: (0, 0))
    return pl.pallas_call(
        _uv_body,
        grid=(N // NB,),
        in_specs=[pl.BlockSpec((NB, D), lambda i: (i, 0)),
                  w((D, D)), w((D, D)), w((1, D))],
        out_specs=[pl.BlockSpec((NB, D), lambda i: (i, 0)),
                   pl.BlockSpec((NB, D), lambda i: (i, 0))],
        out_shape=[jax.ShapeDtypeStruct((N, D), jnp.float32),
                   jax.ShapeDtypeStruct((N, D), jnp.float32)],
    )(x, a, b, b1e)


def _edge_body(gs, e, c, w2, b2, g, bt, out):
    ev = e[...]
    h = jnp.maximum(gs[...] + ev @ c[...], 0.0)
    y = h @ w2[...] + b2[...]
    out[...] = _ln(y, g[...], bt[...]) + ev


def _tc_edge(gs, e, c, w2, b2, g, bt):
    ecnt = gs.shape[0]
    w = lambda shape: pl.BlockSpec(shape, lambda i: (0, 0))
    eb = pl.BlockSpec((EB, D), lambda i: (i, 0))
    return pl.pallas_call(
        _edge_body,
        grid=(ecnt // EB,),
        in_specs=[eb, eb, w((D, D)), w((D, D)), w((1, D)),
                  w((1, D)), w((1, D))],
        out_specs=eb,
        out_shape=jax.ShapeDtypeStruct((ecnt, D), jnp.float32),
    )(gs, e, c, w2, b2, g, bt)


def _node_body(x, a0, a1, a2, a3, wx, wa, b1, w2, b2, g, bt, out):
    xv = x[...]
    agg = (a0[...] + a1[...]) + (a2[...] + a3[...])
    h = jnp.maximum(xv @ wx[...] + agg @ wa[...] + b1[...], 0.0)
    y = h @ w2[...] + b2[...]
    out[...] = _ln(y, g[...], bt[...]) + xv


def _tc_node(x, aggs_a, aggs_b, wx, wa, b1, w2, b2, g, bt):
    w = lambda shape: pl.BlockSpec(shape, lambda i: (0, 0))
    nb = pl.BlockSpec((NB, D), lambda i: (i, 0))
    nb1 = pl.BlockSpec((NB, D), lambda i: (N // NB + i, 0))
    return pl.pallas_call(
        _node_body,
        grid=(N // NB,),
        in_specs=[nb, nb, nb1, nb, nb1, w((D, D)), w((D, D)), w((1, D)),
                  w((D, D)), w((1, D)), w((1, D)), w((1, D))],
        out_specs=nb,
        out_shape=jax.ShapeDtypeStruct((N, D), jnp.float32),
    )(x, aggs_a, aggs_a, aggs_b, aggs_b, wx, wa, b1, w2, b2, g, bt)


def _dec_body(x, w1, b1, w2, b2, out):
    h = jnp.maximum(x[...] @ w1[...] + b1[...], 0.0)
    out[...] = h @ w2[...] + b2[...]


def _tc_dec(x, w1, b1, w2, b2):
    w = lambda shape: pl.BlockSpec(shape, lambda i: (0, 0))
    return pl.pallas_call(
        _dec_body,
        grid=(N // NB,),
        in_specs=[pl.BlockSpec((NB, D), lambda i: (i, 0)),
                  w((D, D)), w((1, D)), w((D, OUT)), w((1, OUT))],
        out_specs=pl.BlockSpec((NB, OUT), lambda i: (i, 0)),
        out_shape=jax.ShapeDtypeStruct((N, OUT), jnp.float32),
    )(x, w1, b1, w2, b2)


@functools.cache
def _sc_gather_kernel(e_lo, ecnt):
    mesh = plsc.VectorSubcoreMesh(core_axis_name="c", subcore_axis_name="s")
    ew = ecnt // NW

    @functools.partial(
        pl.kernel,
        out_type=jax.ShapeDtypeStruct((ecnt, D), jnp.float32),
        mesh=mesh,
        scratch_types=[
            pltpu.VMEM((GK,), jnp.int32),
            pltpu.VMEM((GK,), jnp.int32),
            pltpu.VMEM((GK,), jnp.int32),
            pltpu.VMEM((GK,), jnp.int32),
            pltpu.VMEM((GK, D), jnp.float32),
            pltpu.VMEM((GK, D), jnp.float32),
            pltpu.VMEM((GK, D), jnp.float32),
            pltpu.VMEM((GK, D), jnp.float32),
            pltpu.SemaphoreType.DMA,
            pltpu.SemaphoreType.DMA,
            pltpu.SemaphoreType.DMA,
            pltpu.SemaphoreType.DMA,
        ],
    )
    def gather(u_hbm, v_hbm, src_hbm, dst_hbm, g_hbm,
               sidx0, sidx1, didx0, didx1, bu0, bu1, bv0, bv1,
               sg0, sg1, sw0, sw1):
        wid = lax.axis_index("s") * NC + lax.axis_index("c")
        base = wid * ew
        CH = ew // GK
        bufs = {0: (sidx0, didx0, bu0, bv0, sg0, sw0),
                1: (sidx1, didx1, bu1, bv1, sg1, sw1)}

        def start_chunk(i, b):
            sidx, didx, bu, bv, sg, _ = bufs[b]
            off = base + i * GK
            pltpu.sync_copy(src_hbm.at[pl.ds(e_lo + off, GK)], sidx)
            pltpu.sync_copy(dst_hbm.at[pl.ds(e_lo + off, GK)], didx)
            pltpu.async_copy(u_hbm.at[sidx], bu, sg)
            pltpu.async_copy(v_hbm.at[didx], bv, sg)

        def add_rows(b):
            _, _, bu, bv, sg, _ = bufs[b]
            pltpu.make_async_copy(g_hbm.at[pl.ds(0, GK)], bu, sg).wait()
            pltpu.make_async_copy(g_hbm.at[pl.ds(0, GK)], bv, sg).wait()

            def row(r, carry):
                for t in range(D // 16):
                    sl = pl.ds(t * 16, 16)
                    bu[r, sl] += bv[r, sl]
                return carry

            lax.fori_loop(0, GK, row, 0)

        def start_write(i, b):
            _, _, bu, _, _, sw = bufs[b]
            off = base + i * GK
            pltpu.async_copy(bu, g_hbm.at[pl.ds(off, GK)], sw)

        def wait_write(b):
            _, _, bu, _, _, sw = bufs[b]
            pltpu.make_async_copy(bu, g_hbm.at[pl.ds(0, GK)], sw).wait()

        start_chunk(0, 0)
        add_rows(0)
        start_write(0, 0)
        start_chunk(1, 1)

        def pair(j, carry):
            i1 = 1 + 2 * j
            add_rows(1)
            start_write(i1, 1)
            wait_write(0)
            start_chunk(i1 + 1, 0)
            i2 = i1 + 1
            add_rows(0)
            start_write(i2, 0)
            wait_write(1)
            start_chunk(i2 + 1, 1)
            return carry

        lax.fori_loop(0, (CH - 2) // 2, pair, 0)
        add_rows(1)
        start_write(CH - 1, 1)
        wait_write(0)
        wait_write(1)

    return gather


def _sc_gather(u, v, src, dst, h):
    return _sc_gather_kernel(ELO[h], EHALF[h])(u, v, src, dst)


@functools.cache
def _sc_scatter_kernel(e_lo, ecnt):
    mesh = plsc.VectorSubcoreMesh(core_axis_name="c", subcore_axis_name="s")
    ew = ecnt // NW

    @functools.partial(
        pl.kernel,
        out_type=jax.ShapeDtypeStruct((NC, NS, ROWS, D), jnp.float32),
        mesh=mesh,
        scratch_types=[
            pltpu.VMEM_SHARED((N, D), jnp.float32),
            pltpu.VMEM((SK, D), jnp.float32),
            pltpu.VMEM((SK, D), jnp.float32),
            pltpu.VMEM((SK,), jnp.int32),
            pltpu.VMEM((SK,), jnp.int32),
            pltpu.SemaphoreType.DMA,
            pltpu.SemaphoreType.DMA,
        ],
    )
    def scatter(e_hbm, dst_hbm, zeros_hbm, out_hbm, shared,
                buf0, buf1, didx0, didx1, sl0, sl1):
        c = lax.axis_index("c")
        s = lax.axis_index("s")
        wid = s * NC + c
        r0 = s * ROWS
        pltpu.sync_copy(zeros_hbm, shared.at[pl.ds(r0, ROWS)])
        plsc.subcore_barrier()
        base = wid * ew
        CH = ew // SK
        bufs = {0: (buf0, didx0, sl0), 1: (buf1, didx1, sl1)}

        def start_load(i, b):
            buf, didx, sl = bufs[b]
            off = base + i * SK
            pltpu.async_copy(dst_hbm.at[pl.ds(e_lo + off, SK)], didx, sl)
            pltpu.async_copy(e_hbm.at[pl.ds(off, SK)], buf, sl)

        def scatter_chunk(b):
            buf, didx, sl = bufs[b]
            pltpu.make_async_copy(dst_hbm.at[pl.ds(0, SK)], didx, sl).wait()
            pltpu.make_async_copy(e_hbm.at[pl.ds(0, SK)], buf, sl).wait()
            pltpu.sync_copy(buf, shared.at[didx], add=True)


        start_load(0, 0)

        def pair(j, carry):
            start_load(2 * j + 1, 1)
            scatter_chunk(0)
            start_load(2 * j + 2, 0)
            scatter_chunk(1)
            return carry

        lax.fori_loop(0, (CH - 2) // 2, pair, 0)
        start_load(CH - 1, 1)
        scatter_chunk(0)
        scatter_chunk(1)
        plsc.subcore_barrier()
        pltpu.sync_copy(shared.at[pl.ds(r0, ROWS)], out_hbm.at[c, s])

    return scatter


def _sc_scatter(e2, dst, zeros_rows, h):
    out = _sc_scatter_kernel(ELO[h], EHALF[h])(e2, dst, zeros_rows)
    return out.reshape(NC * N, D)


def _rowvec(b):
    return b.reshape(1, -1)


def kernel(node_attr, edge_attr, edge_index, params):
    src = edge_index[0]
    dst = edge_index[1]
    p = params
    zeros_rows = jnp.zeros((ROWS, D), jnp.float32)

    ge = p['global_enc']
    gf = _tc_global_mean(node_attr, ge['W1'], _rowvec(ge['b1']),
                         ge['W2'], _rowvec(ge['b2']))

    ne = p['node_enc']
    x = _tc_node_enc(node_attr, gf, ne['W1'][:D], ne['W1'][D:],
                     _rowvec(ne['b1']), ne['W2'], _rowvec(ne['b2']),
                     _rowvec(ne['g']), _rowvec(ne['bt']))

    ee = p['edge_enc']
    ea = _tc_edge_enc(edge_attr, ee['W1'], _rowvec(ee['b1']), ee['W2'],
                      _rowvec(ee['b2']), _rowvec(ee['g']), _rowvec(ee['bt']),
                      0)
    eb = _tc_edge_enc(edge_attr, ee['W1'], _rowvec(ee['b1']), ee['W2'],
                      _rowvec(ee['b2']), _rowvec(ee['g']), _rowvec(ee['bt']),
                      1)

    def stk(fn):
        return jnp.stack([fn(lp) for lp in p['layers']])

    ws = {
        'A': stk(lambda lp: lp['edge_mlp']['W1'][:D]),
        'B': stk(lambda lp: lp['edge_mlp']['W1'][D:2 * D]),
        'C': stk(lambda lp: lp['edge_mlp']['W1'][2 * D:]),
        'b1e': stk(lambda lp: _rowvec(lp['edge_mlp']['b1'])),
        'W2e': stk(lambda lp: lp['edge_mlp']['W2']),
        'b2e': stk(lambda lp: _rowvec(lp['edge_mlp']['b2'])),
        'ge': stk(lambda lp: _rowvec(lp['edge_mlp']['g'])),
        'be': stk(lambda lp: _rowvec(lp['edge_mlp']['bt'])),
        'Wx': stk(lambda lp: lp['node_mlp']['W1'][:D]),
        'Wa': stk(lambda lp: lp['node_mlp']['W1'][D:]),
        'b1n': stk(lambda lp: _rowvec(lp['node_mlp']['b1'])),
        'W2n': stk(lambda lp: lp['node_mlp']['W2']),
        'b2n': stk(lambda lp: _rowvec(lp['node_mlp']['b2'])),
        'gn': stk(lambda lp: _rowvec(lp['node_mlp']['g'])),
        'bn': stk(lambda lp: _rowvec(lp['node_mlp']['bt'])),
    }

    def body(carry, w):
        x, ea, eb = carry
        u, v = _tc_uv(x, w['A'], w['B'], w['b1e'])
        gs_a = _sc_gather(u, v, src, dst, 0)
        gs_b = _sc_gather(u, v, src, dst, 1)
        e2a = _tc_edge(gs_a, ea, w['C'], w['W2e'], w['b2e'],
                       w['ge'], w['be'])
        e2b = _tc_edge(gs_b, eb, w['C'], w['W2e'], w['b2e'],
                       w['ge'], w['be'])
        aggs_a = _sc_scatter(e2a, dst, zeros_rows, 0)
        aggs_b = _sc_scatter(e2b, dst, zeros_rows, 1)
        x2 = _tc_node(x, aggs_a, aggs_b, w['Wx'], w['Wa'], w['b1n'],
                      w['W2n'], w['b2n'], w['gn'], w['bn'])
        return (x2, e2a, e2b), None

    (x, ea, eb), _ = lax.scan(body, (x, ea, eb), ws)

    dec = p['decoder']
    return _tc_dec(x, dec['W1'], _rowvec(dec['b1']), dec['W2'],
                   _rowvec(dec['b2']))
